# baseline (speedup 1.0000x reference)
import functools

import jax
import jax.numpy as jnp
from jax import lax
from jax.experimental import pallas as pl
from jax.experimental.pallas import tpu as pltpu
from jax.experimental.pallas import tpu_sc as plsc

_E = 8
_K = 2
_H = 768
_B = 4
_S = 8192
_BS = 2048

_NW = 32
_SCH = _S // _NW
_TW = _SCH * _B
_GRP = 16


def _logits_body(x_ref, w_ref, b_ref, o0, o1, o2, o3):
    w = w_ref[...]
    bias = b_ref[...]
    outs = (o0, o1, o2, o3)
    for bi in range(_B):
        xb = x_ref[:, bi, :]
        lg = jax.lax.dot_general(
            xb, w, (((1,), (1,)), ((), ())),
            preferred_element_type=jnp.float32) + bias
        outs[bi][...] = lg.T


def _sc_router(t0_hbm, t1_hbm, t2_hbm, t3_hbm, lgT_hbm, affT_hbm, idxT_hbm,
               in_v, lgT_v, affT_v, idxT_v):
    nc = 2
    wid = lax.axis_index("s") * nc + lax.axis_index("c")
    s0 = wid * _SCH
    t0 = s0 * _B
    ins = (t0_hbm, t1_hbm, t2_hbm, t3_hbm)
    for bi in range(_B):
        pltpu.sync_copy(ins[bi].at[:, pl.ds(s0, _SCH)], in_v.at[bi])

    lane = lax.iota(jnp.int32, 16)

    def group(g, carry):
        base = g * _GRP
        for bi in range(_B):
            v = [in_v[bi, e, pl.ds(base, _GRP)] for e in range(_E)]
            best = v[0]
            bidx = jnp.zeros((16,), jnp.int32)
            for e in range(1, _E):
                upd = v[e] > best
                best = jnp.where(upd, v[e], best)
                bidx = jnp.where(upd, e, bidx)
            ninf = jnp.full((16,), -jnp.inf, jnp.float32)
            b2v = jnp.where(bidx == 0, ninf, v[0])
            b2i = jnp.zeros((16,), jnp.int32)
            for e in range(1, _E):
                ve = jnp.where(bidx == e, ninf, v[e])
                upd = ve > b2v
                b2v = jnp.where(upd, ve, b2v)
                b2i = jnp.where(upd, e, b2i)
            ex = [jnp.exp(v[e] - best) for e in range(_E)]
            tot = ex[0]
            for e in range(1, _E):
                tot = tot + ex[e]
            r = jnp.full((16,), 1.0, jnp.float32) / tot
            pos = (base + lane) * _B + bi
            for e in range(_E):
                col = jnp.full((16,), e, jnp.int32)
                plsc.store_scatter(lgT_v, [col, pos], v[e])
                plsc.store_scatter(affT_v, [col, pos], ex[e] * r)
            plsc.store_scatter(idxT_v, [jnp.zeros((16,), jnp.int32), pos],
                               bidx)
            plsc.store_scatter(idxT_v, [jnp.ones((16,), jnp.int32), pos],
                               b2i)
        return carry

    lax.fori_loop(0, _SCH // _GRP, group, 0)

    pltpu.sync_copy(lgT_v, lgT_hbm.at[:, pl.ds(t0, _TW)])
    pltpu.sync_copy(affT_v, affT_hbm.at[:, pl.ds(t0, _TW)])
    pltpu.sync_copy(idxT_v, idxT_hbm.at[:, pl.ds(t0, _TW)])


@jax.jit
def kernel(hidden_states, W, b):
    s = hidden_states.shape[0]
    t = s * _B
    b2 = b.reshape(1, _E)
    slab = jax.ShapeDtypeStruct((_E, s), jnp.float32)
    lgT0, lgT1, lgT2, lgT3 = pl.pallas_call(
        _logits_body,
        grid=(s // _BS,),
        in_specs=[
            pl.BlockSpec((_BS, _B, _H), lambda i: (i, 0, 0)),
            pl.BlockSpec((_E, _H), lambda i: (0, 0)),
            pl.BlockSpec((1, _E), lambda i: (0, 0)),
        ],
        out_specs=[pl.BlockSpec((_E, _BS), lambda i: (0, i))] * _B,
        out_shape=[slab] * _B,
        compiler_params=pltpu.CompilerParams(
            dimension_semantics=("arbitrary",)),
    )(hidden_states, W, b2)

    mesh = plsc.VectorSubcoreMesh(core_axis_name="c", subcore_axis_name="s")
    sc = functools.partial(
        pl.kernel, mesh=mesh,
        compiler_params=pltpu.CompilerParams(needs_layout_passes=False),
        out_type=[
            jax.ShapeDtypeStruct((_E, t), jnp.float32),
            jax.ShapeDtypeStruct((_E, t), jnp.float32),
            jax.ShapeDtypeStruct((_K, t), jnp.int32),
        ],
        scratch_types=[
            pltpu.VMEM((_B, _E, _SCH), jnp.float32),
            pltpu.VMEM((_E, _TW), jnp.float32),
            pltpu.VMEM((_E, _TW), jnp.float32),
            pltpu.VMEM((_K, _TW), jnp.int32),
        ],
    )(_sc_router)
    lgT, affT, idxT = sc(lgT0, lgT1, lgT2, lgT3)
    return (lgT.T, affT.T, idxT.T)


# speedup vs baseline: 1.0524x; 1.0524x over previous
import functools

import jax
import jax.numpy as jnp
from jax import lax
from jax.experimental import pallas as pl
from jax.experimental.pallas import tpu as pltpu
from jax.experimental.pallas import tpu_sc as plsc

_E = 8
_K = 2
_H = 768
_B = 4
_S = 8192
_BS = 1024

_NW = 32
_SCH = _S // _NW
_TW = _SCH * _B
_GRP = 16


def _logits_body(x_ref, w_ref, b_ref, o0, o1, o2, o3):
    w = w_ref[...]
    bias = b_ref[...]
    outs = (o0, o1, o2, o3)
    for bi in range(_B):
        xb = x_ref[:, bi, :]
        lg = jax.lax.dot_general(
            xb, w, (((1,), (1,)), ((), ())),
            preferred_element_type=jnp.float32) + bias
        outs[bi][...] = lg.T


def _sc_router(t0_hbm, t1_hbm, t2_hbm, t3_hbm, lgT_hbm, affT_hbm, idxT_hbm,
               in_v, lgT_v, affT_v, idxT_v):
    nc = 2
    wid = lax.axis_index("s") * nc + lax.axis_index("c")
    s0 = wid * _SCH
    t0 = s0 * _B
    ins = (t0_hbm, t1_hbm, t2_hbm, t3_hbm)
    for bi in range(_B):
        pltpu.sync_copy(ins[bi].at[:, pl.ds(s0, _SCH)], in_v.at[bi])

    lane = lax.iota(jnp.int32, 16)

    def group(g, carry):
        base = g * _GRP
        for bi in range(_B):
            v = [in_v[bi, e, pl.ds(base, _GRP)] for e in range(_E)]
            best = v[0]
            bidx = jnp.zeros((16,), jnp.int32)
            for e in range(1, _E):
                upd = v[e] > best
                best = jnp.where(upd, v[e], best)
                bidx = jnp.where(upd, e, bidx)
            ninf = jnp.full((16,), -jnp.inf, jnp.float32)
            b2v = jnp.where(bidx == 0, ninf, v[0])
            b2i = jnp.zeros((16,), jnp.int32)
            for e in range(1, _E):
                ve = jnp.where(bidx == e, ninf, v[e])
                upd = ve > b2v
                b2v = jnp.where(upd, ve, b2v)
                b2i = jnp.where(upd, e, b2i)
            ex = [jnp.exp(v[e] - best) for e in range(_E)]
            tot = ex[0]
            for e in range(1, _E):
                tot = tot + ex[e]
            r = jnp.full((16,), 1.0, jnp.float32) / tot
            pos = (base + lane) * _B + bi
            for e in range(_E):
                col = jnp.full((16,), e, jnp.int32)
                plsc.store_scatter(lgT_v, [col, pos], v[e])
                plsc.store_scatter(affT_v, [col, pos], ex[e] * r)
            plsc.store_scatter(idxT_v, [jnp.zeros((16,), jnp.int32), pos],
                               bidx)
            plsc.store_scatter(idxT_v, [jnp.ones((16,), jnp.int32), pos],
                               b2i)
        return carry

    lax.fori_loop(0, _SCH // _GRP, group, 0)

    pltpu.sync_copy(lgT_v, lgT_hbm.at[:, pl.ds(t0, _TW)])
    pltpu.sync_copy(affT_v, affT_hbm.at[:, pl.ds(t0, _TW)])
    pltpu.sync_copy(idxT_v, idxT_hbm.at[:, pl.ds(t0, _TW)])


@jax.jit
def kernel(hidden_states, W, b):
    s = hidden_states.shape[0]
    t = s * _B
    b2 = b.reshape(1, _E)
    slab = jax.ShapeDtypeStruct((_E, s), jnp.float32)
    lgT0, lgT1, lgT2, lgT3 = pl.pallas_call(
        _logits_body,
        grid=(s // _BS,),
        in_specs=[
            pl.BlockSpec((_BS, _B, _H), lambda i: (i, 0, 0)),
            pl.BlockSpec((_E, _H), lambda i: (0, 0)),
            pl.BlockSpec((1, _E), lambda i: (0, 0)),
        ],
        out_specs=[pl.BlockSpec((_E, _BS), lambda i: (0, i))] * _B,
        out_shape=[slab] * _B,
        compiler_params=pltpu.CompilerParams(
            dimension_semantics=("arbitrary",)),
    )(hidden_states, W, b2)

    mesh = plsc.VectorSubcoreMesh(core_axis_name="c", subcore_axis_name="s")
    sc = functools.partial(
        pl.kernel, mesh=mesh,
        compiler_params=pltpu.CompilerParams(needs_layout_passes=False),
        out_type=[
            jax.ShapeDtypeStruct((_E, t), jnp.float32),
            jax.ShapeDtypeStruct((_E, t), jnp.float32),
            jax.ShapeDtypeStruct((_K, t), jnp.int32),
        ],
        scratch_types=[
            pltpu.VMEM((_B, _E, _SCH), jnp.float32),
            pltpu.VMEM((_E, _TW), jnp.float32),
            pltpu.VMEM((_E, _TW), jnp.float32),
            pltpu.VMEM((_K, _TW), jnp.int32),
        ],
    )(_sc_router)
    lgT, affT, idxT = sc(lgT0, lgT1, lgT2, lgT3)
    return (lgT.T, affT.T, idxT.T)
